# baseline (speedup 1.0000x reference)
import functools

import jax
import jax.numpy as jnp
from jax import lax
from jax.experimental import pallas as pl
from jax.experimental.pallas import tpu as pltpu
from jax.experimental.pallas import tpu_sc as plsc

_NEG = -1e30
_INV_SQRT2 = 0.7071067811865476

_S = 2048
_H = 1024
_E = 8
_I = 4096
_T = 256
_NT = (2 * _S) // _T + (_E - 1)
_NROWS = _NT * _T
_XROWS = _NROWS + _T
_MAXT = (2 * _S // 2) // _T

_NW = 32
_NTOK = _S // _NW
_CHUNK = 32


def _gelu_exact(v):
    return 0.5 * v * (1.0 + jax.lax.erf(v * _INV_SQRT2))


def _route_body(x_ref, gw_ref, w0_ref, w1_ref, d0_ref, d1_ref,
                tstart_ref, ntile_ref):
    logits = jax.lax.dot_general(
        x_ref[...], gw_ref[...], (((1,), (1,)), ((), ())),
        preferred_element_type=jnp.float32)
    s, e = logits.shape
    col = jax.lax.broadcasted_iota(jnp.int32, (s, e), 1)
    a1 = jnp.argmax(logits, axis=-1)
    oh1 = col == a1[:, None]
    m1 = jnp.max(logits, axis=-1, keepdims=True)
    masked = jnp.where(oh1, _NEG, logits)
    a2 = jnp.argmax(masked, axis=-1)
    oh2 = col == a2[:, None]
    m2 = jnp.max(masked, axis=-1, keepdims=True)
    z = jnp.exp(m2 - m1)
    w0_ref[...] = 1.0 / (1.0 + z)
    w1_ref[...] = z / (1.0 + z)

    oh = jnp.concatenate([oh1, oh2], axis=0).astype(jnp.int32)
    csum = oh
    k = 1
    while k < 2 * s:
        csum = csum + jnp.concatenate(
            [jnp.zeros((k, e), jnp.int32), csum[:-k]], axis=0)
        k *= 2
    counts = csum[2 * s - 1:2 * s, :]
    padded = ((counts + (_T - 1)) // _T) * _T
    pc = padded
    k = 1
    while k < e:
        pc = pc + jnp.concatenate(
            [jnp.zeros((1, k), jnp.int32), pc[:, :-k]], axis=1)
        k *= 2
    base = pc - padded

    rank = jnp.sum(oh * csum, axis=1, keepdims=True) - 1
    belem = jnp.sum(oh * base, axis=1, keepdims=True)
    dest = belem + rank
    d0_ref[...] = dest[:s]
    d1_ref[...] = dest[s:]

    ones = jnp.ones((2 * s, 1), jnp.float32)
    counts_t = jax.lax.dot_general(
        oh.astype(jnp.float32), ones, (((0,), (0,)), ((), ())),
        preferred_element_type=jnp.float32).astype(jnp.int32)
    padded_t = ((counts_t + (_T - 1)) // _T) * _T
    pc_t = padded_t
    k = 1
    while k < e:
        pc_t = pc_t + jnp.concatenate(
            [jnp.zeros((k, 1), jnp.int32), pc_t[:-k]], axis=0)
        k *= 2
    base_t = pc_t - padded_t
    tstart_ref[...] = base_t // _T
    ntile_ref[...] = padded_t // _T


def _ffn_body(tstart_ref, ntile_ref, xs_ref, w1_ref, b1_ref, w2_ref, b2_ref,
              ys_ref, xb0, xb1, ob0, ob1, sin, sout):
    ei = pl.program_id(0)
    ts = tstart_ref[ei]
    nt = ntile_ref[ei]
    xbufs = (xb0, xb1)
    obufs = (ob0, ob1)

    def cp_in(k, b):
        return pltpu.make_async_copy(
            xs_ref.at[pl.ds((ts + k) * _T, _T), :], xbufs[b], sin.at[b])

    def cp_out(k, b):
        return pltpu.make_async_copy(
            obufs[b], ys_ref.at[pl.ds((ts + k) * _T, _T), :], sout.at[b])

    @pl.when(nt > 0)
    def _prologue():
        cp_in(0, 0).start()

    for k in range(_MAXT):
        bsel = k % 2

        @pl.when(k < nt)
        def _tile(k=k, bsel=bsel):
            @pl.when(k + 1 < nt)
            def _prefetch():
                cp_in(k + 1, 1 - bsel).start()

            cp_in(k, bsel).wait()
            xb = xbufs[bsel][...].astype(jnp.bfloat16)
            h = jax.lax.dot_general(
                xb, w1_ref[0], (((1,), (1,)), ((), ())),
                preferred_element_type=jnp.float32)
            h = _gelu_exact(h + b1_ref[0, 0][None, :])
            o = jax.lax.dot_general(
                h.astype(jnp.bfloat16), w2_ref[0], (((1,), (1,)), ((), ())),
                preferred_element_type=jnp.float32)
            if k >= 2:
                cp_out(k - 2, bsel).wait()
            obufs[bsel][...] = o + b2_ref[0, 0][None, :]
            cp_out(k, bsel).start()

    for p in range(2):
        @pl.when(jnp.logical_and(nt >= 1, (nt - 1) % 2 == p))
        def _drain_last(p=p):
            cp_out(0, p).wait()

        @pl.when(jnp.logical_and(nt >= 2, nt % 2 == p))
        def _drain_prev(p=p):
            cp_out(0, p).wait()


def _comb_body(r0_ref, r1_ref, w0_ref, w1_ref, o_ref):
    o_ref[...] = w0_ref[...] * r0_ref[...] + w1_ref[...] * r1_ref[...]


@functools.lru_cache(maxsize=None)
def _make_dispatch_sc():
    mesh = plsc.VectorSubcoreMesh(core_axis_name="c", subcore_axis_name="s")

    @functools.partial(
        pl.kernel, mesh=mesh,
        out_type=jax.ShapeDtypeStruct((_NROWS, _H), jnp.float32),
        scratch_types=[
            pltpu.VMEM((_NTOK, _H), jnp.float32),
            pltpu.VMEM((_NTOK,), jnp.int32),
            pltpu.VMEM((_NTOK,), jnp.int32),
            pltpu.SemaphoreType.DMA,
        ])
    def _dispatch_sc(x_hbm, d0_hbm, d1_hbm, xs_hbm, xbuf, i0, i1, sem):
        wid = lax.axis_index("s") * 2 + lax.axis_index("c")
        base = wid * _NTOK
        pltpu.sync_copy(x_hbm.at[pl.ds(base, _NTOK)], xbuf)
        pltpu.sync_copy(d0_hbm.at[pl.ds(base, _NTOK)], i0)
        pltpu.sync_copy(d1_hbm.at[pl.ds(base, _NTOK)], i1)
        pltpu.async_copy(xbuf, xs_hbm.at[i0], sem).wait()
        pltpu.async_copy(xbuf, xs_hbm.at[i1], sem).wait()

    return _dispatch_sc


@functools.lru_cache(maxsize=None)
def _make_combine_sc():
    mesh = plsc.VectorSubcoreMesh(core_axis_name="c", subcore_axis_name="s")

    @functools.partial(
        pl.kernel, mesh=mesh,
        out_type=(jax.ShapeDtypeStruct((_S, _H), jnp.float32),
                  jax.ShapeDtypeStruct((_S, _H), jnp.float32)),
        scratch_types=[
            pltpu.VMEM((_CHUNK, _H), jnp.float32),
            pltpu.VMEM((_CHUNK, _H), jnp.float32),
            pltpu.VMEM((_CHUNK,), jnp.int32),
            pltpu.VMEM((_CHUNK,), jnp.int32),
            pltpu.SemaphoreType.DMA,
        ])
    def _combine_sc(ys_hbm, d0_hbm, d1_hbm, r0_hbm, r1_hbm,
                    b0, b1, i0, i1, sem):
        wid = lax.axis_index("s") * 2 + lax.axis_index("c")
        for c in range(_NTOK // _CHUNK):
            bc = wid * _NTOK + c * _CHUNK
            pltpu.sync_copy(d0_hbm.at[pl.ds(bc, _CHUNK)], i0)
            pltpu.sync_copy(d1_hbm.at[pl.ds(bc, _CHUNK)], i1)
            pltpu.async_copy(ys_hbm.at[i0], b0, sem).wait()
            pltpu.async_copy(ys_hbm.at[i1], b1, sem).wait()
            pltpu.sync_copy(b0, r0_hbm.at[pl.ds(bc, _CHUNK)])
            pltpu.sync_copy(b1, r1_hbm.at[pl.ds(bc, _CHUNK)])

    return _combine_sc


def kernel(x, gate_w, w1, b1, w2, b2):
    b, s, hd = x.shape
    e, i, _ = w1.shape
    xf = x.reshape(s, hd)

    w0s, w1s, d0, d1, tstart, ntile = pl.pallas_call(
        _route_body,
        grid=(1,),
        in_specs=[
            pl.BlockSpec((s, hd), lambda _: (0, 0)),
            pl.BlockSpec((e, hd), lambda _: (0, 0)),
        ],
        out_specs=[
            pl.BlockSpec((s, 1), lambda _: (0, 0)),
            pl.BlockSpec((s, 1), lambda _: (0, 0)),
            pl.BlockSpec((s, 1), lambda _: (0, 0)),
            pl.BlockSpec((s, 1), lambda _: (0, 0)),
            pl.BlockSpec((e, 1), lambda _: (0, 0)),
            pl.BlockSpec((e, 1), lambda _: (0, 0)),
        ],
        out_shape=[
            jax.ShapeDtypeStruct((s, 1), jnp.float32),
            jax.ShapeDtypeStruct((s, 1), jnp.float32),
            jax.ShapeDtypeStruct((s, 1), jnp.int32),
            jax.ShapeDtypeStruct((s, 1), jnp.int32),
            jax.ShapeDtypeStruct((e, 1), jnp.int32),
            jax.ShapeDtypeStruct((e, 1), jnp.int32),
        ],
    )(xf, gate_w)

    d0f = d0.reshape(s)
    d1f = d1.reshape(s)
    xs = _make_dispatch_sc()(xf, d0f, d1f)

    b1r = b1.reshape(e, 1, i)
    b2r = b2.reshape(e, 1, hd)
    w1b = w1.astype(jnp.bfloat16)
    w2b = w2.astype(jnp.bfloat16)

    ys = pl.pallas_call(
        _ffn_body,
        grid_spec=pltpu.PrefetchScalarGridSpec(
            num_scalar_prefetch=2,
            grid=(e,),
            in_specs=[
                pl.BlockSpec(memory_space=pl.ANY),
                pl.BlockSpec((1, i, hd), lambda ei, ts, nt: (ei, 0, 0)),
                pl.BlockSpec((1, 1, i), lambda ei, ts, nt: (ei, 0, 0)),
                pl.BlockSpec((1, hd, i), lambda ei, ts, nt: (ei, 0, 0)),
                pl.BlockSpec((1, 1, hd), lambda ei, ts, nt: (ei, 0, 0)),
            ],
            out_specs=pl.BlockSpec(memory_space=pl.ANY),
            scratch_shapes=[
                pltpu.VMEM((_T, hd), jnp.float32),
                pltpu.VMEM((_T, hd), jnp.float32),
                pltpu.VMEM((_T, hd), jnp.float32),
                pltpu.VMEM((_T, hd), jnp.float32),
                pltpu.SemaphoreType.DMA((2,)),
                pltpu.SemaphoreType.DMA((2,)),
            ],
        ),
        out_shape=jax.ShapeDtypeStruct((_NROWS, hd), jnp.float32),
        compiler_params=pltpu.CompilerParams(
            dimension_semantics=("arbitrary",)),
    )(tstart.reshape(e), ntile.reshape(e), xs, w1b, b1r, w2b, b2r)

    r0, r1 = _make_combine_sc()(ys, d0f, d1f)

    ts = 512
    out = pl.pallas_call(
        _comb_body,
        grid=(s // ts,),
        in_specs=[
            pl.BlockSpec((ts, hd), lambda si: (si, 0)),
            pl.BlockSpec((ts, hd), lambda si: (si, 0)),
            pl.BlockSpec((ts, 1), lambda si: (si, 0)),
            pl.BlockSpec((ts, 1), lambda si: (si, 0)),
        ],
        out_specs=pl.BlockSpec((ts, hd), lambda si: (si, 0)),
        out_shape=jax.ShapeDtypeStruct((s, hd), jnp.float32),
    )(r0, r1, w0s, w1s)
    return out.reshape(b, s, hd)


# speedup vs baseline: 1.2584x; 1.2584x over previous
import functools

import jax
import jax.numpy as jnp
from jax import lax
from jax.experimental import pallas as pl
from jax.experimental.pallas import tpu as pltpu
from jax.experimental.pallas import tpu_sc as plsc

_NEG = -1e30
_INV_SQRT2 = 0.7071067811865476

_S = 2048
_H = 1024
_E = 8
_I = 4096
_T = 512
_NT = (2 * _S) // _T + (_E - 1)
_NROWS = _NT * _T
_MAXT = _S // _T
_ICS = 2048
_NIC = _I // _ICS

_NW = 32
_NTOK = _S // _NW
_CHUNK = 32


def _gelu_exact(v):
    return 0.5 * v * (1.0 + jax.lax.erf(v * _INV_SQRT2))


def _route_body(x_ref, gw_ref, w0_ref, w1_ref, d0_ref, d1_ref,
                tstart_ref, ntile_ref):
    logits = jax.lax.dot_general(
        x_ref[...], gw_ref[...], (((1,), (1,)), ((), ())),
        preferred_element_type=jnp.float32)
    s, e = logits.shape
    col = jax.lax.broadcasted_iota(jnp.int32, (s, e), 1)
    a1 = jnp.argmax(logits, axis=-1)
    oh1 = col == a1[:, None]
    m1 = jnp.max(logits, axis=-1, keepdims=True)
    masked = jnp.where(oh1, _NEG, logits)
    a2 = jnp.argmax(masked, axis=-1)
    oh2 = col == a2[:, None]
    m2 = jnp.max(masked, axis=-1, keepdims=True)
    z = jnp.exp(m2 - m1)
    w0_ref[...] = 1.0 / (1.0 + z)
    w1_ref[...] = z / (1.0 + z)

    oh = jnp.concatenate([oh1, oh2], axis=0).astype(jnp.int32)
    csum = oh
    k = 1
    while k < 2 * s:
        csum = csum + jnp.concatenate(
            [jnp.zeros((k, e), jnp.int32), csum[:-k]], axis=0)
        k *= 2
    counts = csum[2 * s - 1:2 * s, :]
    padded = ((counts + (_T - 1)) // _T) * _T
    pc = padded
    k = 1
    while k < e:
        pc = pc + jnp.concatenate(
            [jnp.zeros((1, k), jnp.int32), pc[:, :-k]], axis=1)
        k *= 2
    base = pc - padded

    rank = jnp.sum(oh * csum, axis=1, keepdims=True) - 1
    belem = jnp.sum(oh * base, axis=1, keepdims=True)
    dest = belem + rank
    d0_ref[...] = dest[:s]
    d1_ref[...] = dest[s:]

    ones = jnp.ones((2 * s, 1), jnp.float32)
    counts_t = jax.lax.dot_general(
        oh.astype(jnp.float32), ones, (((0,), (0,)), ((), ())),
        preferred_element_type=jnp.float32).astype(jnp.int32)
    padded_t = ((counts_t + (_T - 1)) // _T) * _T
    pc_t = padded_t
    k = 1
    while k < e:
        pc_t = pc_t + jnp.concatenate(
            [jnp.zeros((k, 1), jnp.int32), pc_t[:-k]], axis=0)
        k *= 2
    base_t = pc_t - padded_t
    tstart_ref[...] = base_t // _T
    ntile_ref[...] = padded_t // _T


def _ffn_body(tstart_ref, ntile_ref, xs_ref, w1_ref, b1_ref, w2_ref, b2_ref,
              ys_ref, xb0, xb1, acc, sin, sout):
    ei = pl.program_id(0)
    ic = pl.program_id(1)
    ts = tstart_ref[ei]
    nt = ntile_ref[ei]
    xbufs = (xb0, xb1)

    def cp_in(k, b):
        return pltpu.make_async_copy(
            xs_ref.at[pl.ds((ts + k) * _T, _T), :], xbufs[b], sin.at[b])

    def cp_out(k):
        return pltpu.make_async_copy(
            acc.at[k], ys_ref.at[pl.ds((ts + k) * _T, _T), :], sout.at[k])

    @pl.when(nt > 0)
    def _prologue():
        cp_in(0, 0).start()

    for k in range(_MAXT):
        bsel = k % 2

        @pl.when(k < nt)
        def _tile(k=k, bsel=bsel):
            @pl.when(k + 1 < nt)
            def _prefetch():
                cp_in(k + 1, 1 - bsel).start()

            cp_in(k, bsel).wait()
            h = jax.lax.dot_general(
                xbufs[bsel][...], w1_ref[0], (((1,), (1,)), ((), ())),
                preferred_element_type=jnp.float32)
            h = _gelu_exact(h + b1_ref[0, 0, 0, :][None, :])
            o = jax.lax.dot_general(
                h, w2_ref[0], (((1,), (1,)), ((), ())),
                preferred_element_type=jnp.float32)

            @pl.when(ic == 0)
            def _first():
                acc[k] = o + b2_ref[0, 0][None, :]

            @pl.when(ic > 0)
            def _rest():
                acc[k] += o

            if _NIC > 1:
                @pl.when(ic == _NIC - 1)
                def _writeback():
                    cp_out(k).start()
            else:
                cp_out(k).start()

    @pl.when(ic == _NIC - 1)
    def _drain():
        for k in range(_MAXT):
            @pl.when(k < nt)
            def _wait_out(k=k):
                cp_out(k).wait()


def _comb_body(r0_ref, r1_ref, w0_ref, w1_ref, o_ref):
    o_ref[...] = w0_ref[...] * r0_ref[...] + w1_ref[...] * r1_ref[...]


@functools.lru_cache(maxsize=None)
def _make_dispatch_sc():
    mesh = plsc.VectorSubcoreMesh(core_axis_name="c", subcore_axis_name="s")

    @functools.partial(
        pl.kernel, mesh=mesh,
        out_type=jax.ShapeDtypeStruct((_NROWS, _H), jnp.float32),
        scratch_types=[
            pltpu.VMEM((_NTOK, _H), jnp.float32),
            pltpu.VMEM((_NTOK,), jnp.int32),
            pltpu.VMEM((_NTOK,), jnp.int32),
            pltpu.SemaphoreType.DMA,
        ])
    def _dispatch_sc(x_hbm, d0_hbm, d1_hbm, xs_hbm, xbuf, i0, i1, sem):
        wid = lax.axis_index("s") * 2 + lax.axis_index("c")
        base = wid * _NTOK
        pltpu.sync_copy(x_hbm.at[pl.ds(base, _NTOK)], xbuf)
        pltpu.sync_copy(d0_hbm.at[pl.ds(base, _NTOK)], i0)
        pltpu.sync_copy(d1_hbm.at[pl.ds(base, _NTOK)], i1)
        pltpu.async_copy(xbuf, xs_hbm.at[i0], sem).wait()
        pltpu.async_copy(xbuf, xs_hbm.at[i1], sem).wait()

    return _dispatch_sc


@functools.lru_cache(maxsize=None)
def _make_combine_sc():
    mesh = plsc.VectorSubcoreMesh(core_axis_name="c", subcore_axis_name="s")

    @functools.partial(
        pl.kernel, mesh=mesh,
        out_type=(jax.ShapeDtypeStruct((_S, _H), jnp.float32),
                  jax.ShapeDtypeStruct((_S, _H), jnp.float32)),
        scratch_types=[
            pltpu.VMEM((_CHUNK, _H), jnp.float32),
            pltpu.VMEM((_CHUNK, _H), jnp.float32),
            pltpu.VMEM((_CHUNK,), jnp.int32),
            pltpu.VMEM((_CHUNK,), jnp.int32),
            pltpu.SemaphoreType.DMA,
        ])
    def _combine_sc(ys_hbm, d0_hbm, d1_hbm, r0_hbm, r1_hbm,
                    b0, b1, i0, i1, sem):
        wid = lax.axis_index("s") * 2 + lax.axis_index("c")
        for c in range(_NTOK // _CHUNK):
            bc = wid * _NTOK + c * _CHUNK
            pltpu.sync_copy(d0_hbm.at[pl.ds(bc, _CHUNK)], i0)
            pltpu.sync_copy(d1_hbm.at[pl.ds(bc, _CHUNK)], i1)
            pltpu.async_copy(ys_hbm.at[i0], b0, sem).wait()
            pltpu.async_copy(ys_hbm.at[i1], b1, sem).wait()
            pltpu.sync_copy(b0, r0_hbm.at[pl.ds(bc, _CHUNK)])
            pltpu.sync_copy(b1, r1_hbm.at[pl.ds(bc, _CHUNK)])

    return _combine_sc


def kernel(x, gate_w, w1, b1, w2, b2):
    b, s, hd = x.shape
    e, i, _ = w1.shape
    xf = x.reshape(s, hd)

    w0s, w1s, d0, d1, tstart, ntile = pl.pallas_call(
        _route_body,
        grid=(1,),
        in_specs=[
            pl.BlockSpec((s, hd), lambda _: (0, 0)),
            pl.BlockSpec((e, hd), lambda _: (0, 0)),
        ],
        out_specs=[
            pl.BlockSpec((s, 1), lambda _: (0, 0)),
            pl.BlockSpec((s, 1), lambda _: (0, 0)),
            pl.BlockSpec((s, 1), lambda _: (0, 0)),
            pl.BlockSpec((s, 1), lambda _: (0, 0)),
            pl.BlockSpec((e, 1), lambda _: (0, 0)),
            pl.BlockSpec((e, 1), lambda _: (0, 0)),
        ],
        out_shape=[
            jax.ShapeDtypeStruct((s, 1), jnp.float32),
            jax.ShapeDtypeStruct((s, 1), jnp.float32),
            jax.ShapeDtypeStruct((s, 1), jnp.int32),
            jax.ShapeDtypeStruct((s, 1), jnp.int32),
            jax.ShapeDtypeStruct((e, 1), jnp.int32),
            jax.ShapeDtypeStruct((e, 1), jnp.int32),
        ],
    )(xf, gate_w)

    d0f = d0.reshape(s)
    d1f = d1.reshape(s)
    xs = _make_dispatch_sc()(xf, d0f, d1f)

    b1r = b1.reshape(e, _NIC, 1, _ICS)
    b2r = b2.reshape(e, 1, hd)

    ys = pl.pallas_call(
        _ffn_body,
        grid_spec=pltpu.PrefetchScalarGridSpec(
            num_scalar_prefetch=2,
            grid=(e, _NIC),
            in_specs=[
                pl.BlockSpec(memory_space=pl.ANY),
                pl.BlockSpec((1, _ICS, hd),
                             lambda ei, ic, ts, nt: (ei, ic, 0)),
                pl.BlockSpec((1, 1, 1, _ICS),
                             lambda ei, ic, ts, nt: (ei, ic, 0, 0)),
                pl.BlockSpec((1, hd, _ICS),
                             lambda ei, ic, ts, nt: (ei, 0, ic)),
                pl.BlockSpec((1, 1, hd),
                             lambda ei, ic, ts, nt: (ei, 0, 0)),
            ],
            out_specs=pl.BlockSpec(memory_space=pl.ANY),
            scratch_shapes=[
                pltpu.VMEM((_T, hd), jnp.float32),
                pltpu.VMEM((_T, hd), jnp.float32),
                pltpu.VMEM((_MAXT, _T, hd), jnp.float32),
                pltpu.SemaphoreType.DMA((2,)),
                pltpu.SemaphoreType.DMA((_MAXT,)),
            ],
        ),
        out_shape=jax.ShapeDtypeStruct((_NROWS, hd), jnp.float32),
        compiler_params=pltpu.CompilerParams(
            dimension_semantics=("arbitrary", "arbitrary")),
    )(tstart.reshape(e), ntile.reshape(e), xs, w1, b1r, w2, b2r)

    r0, r1 = _make_combine_sc()(ys, d0f, d1f)

    ts = 512
    out = pl.pallas_call(
        _comb_body,
        grid=(s // ts,),
        in_specs=[
            pl.BlockSpec((ts, hd), lambda si: (si, 0)),
            pl.BlockSpec((ts, hd), lambda si: (si, 0)),
            pl.BlockSpec((ts, 1), lambda si: (si, 0)),
            pl.BlockSpec((ts, 1), lambda si: (si, 0)),
        ],
        out_specs=pl.BlockSpec((ts, hd), lambda si: (si, 0)),
        out_shape=jax.ShapeDtypeStruct((s, hd), jnp.float32),
    )(r0, r1, w0s, w1s)
    return out.reshape(b, s, hd)


# speedup vs baseline: 1.2988x; 1.0321x over previous
import functools

import jax
import jax.numpy as jnp
from jax import lax
from jax.experimental import pallas as pl
from jax.experimental.pallas import tpu as pltpu
from jax.experimental.pallas import tpu_sc as plsc

_NEG = -1e30
_INV_SQRT2 = 0.7071067811865476

_S = 2048
_H = 1024
_E = 8
_I = 4096
_T = 256
_NT = (2 * _S) // _T + (_E - 1)
_NROWS = _NT * _T
_MAXT = _S // _T
_ICS = 2048
_NIC = _I // _ICS

_NW = 32
_NTOK = _S // _NW
_CHUNK = 32


def _gelu_exact(v):
    return 0.5 * v * (1.0 + jax.lax.erf(v * _INV_SQRT2))


def _route_body(x_ref, gw_ref, w0_ref, w1_ref, d0_ref, d1_ref,
                tstart_ref, ntile_ref):
    logits = jax.lax.dot_general(
        x_ref[...], gw_ref[...], (((1,), (1,)), ((), ())),
        preferred_element_type=jnp.float32)
    s, e = logits.shape
    col = jax.lax.broadcasted_iota(jnp.int32, (s, e), 1)
    a1 = jnp.argmax(logits, axis=-1)
    oh1 = col == a1[:, None]
    m1 = jnp.max(logits, axis=-1, keepdims=True)
    masked = jnp.where(oh1, _NEG, logits)
    a2 = jnp.argmax(masked, axis=-1)
    oh2 = col == a2[:, None]
    m2 = jnp.max(masked, axis=-1, keepdims=True)
    z = jnp.exp(m2 - m1)
    w0_ref[...] = 1.0 / (1.0 + z)
    w1_ref[...] = z / (1.0 + z)

    oh = jnp.concatenate([oh1, oh2], axis=0).astype(jnp.int32)
    csum = oh
    k = 1
    while k < 2 * s:
        csum = csum + jnp.concatenate(
            [jnp.zeros((k, e), jnp.int32), csum[:-k]], axis=0)
        k *= 2
    counts = csum[2 * s - 1:2 * s, :]
    padded = ((counts + (_T - 1)) // _T) * _T
    pc = padded
    k = 1
    while k < e:
        pc = pc + jnp.concatenate(
            [jnp.zeros((1, k), jnp.int32), pc[:, :-k]], axis=1)
        k *= 2
    base = pc - padded

    rank = jnp.sum(oh * csum, axis=1, keepdims=True) - 1
    belem = jnp.sum(oh * base, axis=1, keepdims=True)
    dest = belem + rank
    d0_ref[...] = dest[:s]
    d1_ref[...] = dest[s:]

    ones = jnp.ones((2 * s, 1), jnp.float32)
    counts_t = jax.lax.dot_general(
        oh.astype(jnp.float32), ones, (((0,), (0,)), ((), ())),
        preferred_element_type=jnp.float32).astype(jnp.int32)
    padded_t = ((counts_t + (_T - 1)) // _T) * _T
    pc_t = padded_t
    k = 1
    while k < e:
        pc_t = pc_t + jnp.concatenate(
            [jnp.zeros((k, 1), jnp.int32), pc_t[:-k]], axis=0)
        k *= 2
    base_t = pc_t - padded_t
    tstart_ref[...] = base_t // _T
    ntile_ref[...] = padded_t // _T


def _ffn_body(tstart_ref, ntile_ref, xs_ref, w1_ref, b1_ref, w2_ref, b2_ref,
              ys_ref, xb0, xb1, acc, sin, sout):
    ei = pl.program_id(0)
    ic = pl.program_id(1)
    ts = tstart_ref[ei]
    nt = ntile_ref[ei]
    xbufs = (xb0, xb1)

    def cp_in(k, b):
        return pltpu.make_async_copy(
            xs_ref.at[pl.ds((ts + k) * _T, _T), :], xbufs[b], sin.at[b])

    def cp_out(k):
        return pltpu.make_async_copy(
            acc.at[k], ys_ref.at[pl.ds((ts + k) * _T, _T), :], sout.at[k])

    @pl.when(nt > 0)
    def _prologue():
        cp_in(0, 0).start()

    for k in range(_MAXT):
        bsel = k % 2

        @pl.when(k < nt)
        def _tile(k=k, bsel=bsel):
            @pl.when(k + 1 < nt)
            def _prefetch():
                cp_in(k + 1, 1 - bsel).start()

            cp_in(k, bsel).wait()
            h = jax.lax.dot_general(
                xbufs[bsel][...], w1_ref[0], (((1,), (1,)), ((), ())),
                preferred_element_type=jnp.float32)
            h = _gelu_exact(h + b1_ref[0, 0, 0, :][None, :])
            o = jax.lax.dot_general(
                h, w2_ref[0], (((1,), (1,)), ((), ())),
                preferred_element_type=jnp.float32)

            @pl.when(ic == 0)
            def _first():
                acc[k] = o + b2_ref[0, 0][None, :]

            @pl.when(ic > 0)
            def _rest():
                acc[k] += o

            if _NIC > 1:
                @pl.when(ic == _NIC - 1)
                def _writeback():
                    cp_out(k).start()
            else:
                cp_out(k).start()

    @pl.when(ic == _NIC - 1)
    def _drain():
        for k in range(_MAXT):
            @pl.when(k < nt)
            def _wait_out(k=k):
                cp_out(k).wait()


def _comb_body(r0_ref, r1_ref, w0_ref, w1_ref, o_ref):
    o_ref[...] = w0_ref[...] * r0_ref[...] + w1_ref[...] * r1_ref[...]


@functools.lru_cache(maxsize=None)
def _make_dispatch_sc():
    mesh = plsc.VectorSubcoreMesh(core_axis_name="c", subcore_axis_name="s")

    @functools.partial(
        pl.kernel, mesh=mesh,
        out_type=jax.ShapeDtypeStruct((_NROWS, _H), jnp.float32),
        scratch_types=[
            pltpu.VMEM((_NTOK, _H), jnp.float32),
            pltpu.VMEM((_NTOK,), jnp.int32),
            pltpu.VMEM((_NTOK,), jnp.int32),
            pltpu.SemaphoreType.DMA,
        ])
    def _dispatch_sc(x_hbm, d0_hbm, d1_hbm, xs_hbm, xbuf, i0, i1, sem):
        wid = lax.axis_index("s") * 2 + lax.axis_index("c")
        base = wid * _NTOK
        pltpu.sync_copy(x_hbm.at[pl.ds(base, _NTOK)], xbuf)
        pltpu.sync_copy(d0_hbm.at[pl.ds(base, _NTOK)], i0)
        pltpu.sync_copy(d1_hbm.at[pl.ds(base, _NTOK)], i1)
        pltpu.async_copy(xbuf, xs_hbm.at[i0], sem).wait()
        pltpu.async_copy(xbuf, xs_hbm.at[i1], sem).wait()

    return _dispatch_sc


@functools.lru_cache(maxsize=None)
def _make_combine_sc():
    mesh = plsc.VectorSubcoreMesh(core_axis_name="c", subcore_axis_name="s")

    @functools.partial(
        pl.kernel, mesh=mesh,
        out_type=(jax.ShapeDtypeStruct((_S, _H), jnp.float32),
                  jax.ShapeDtypeStruct((_S, _H), jnp.float32)),
        scratch_types=[
            pltpu.VMEM((_CHUNK, _H), jnp.float32),
            pltpu.VMEM((_CHUNK, _H), jnp.float32),
            pltpu.VMEM((_CHUNK,), jnp.int32),
            pltpu.VMEM((_CHUNK,), jnp.int32),
            pltpu.SemaphoreType.DMA,
        ])
    def _combine_sc(ys_hbm, d0_hbm, d1_hbm, r0_hbm, r1_hbm,
                    b0, b1, i0, i1, sem):
        wid = lax.axis_index("s") * 2 + lax.axis_index("c")
        for c in range(_NTOK // _CHUNK):
            bc = wid * _NTOK + c * _CHUNK
            pltpu.sync_copy(d0_hbm.at[pl.ds(bc, _CHUNK)], i0)
            pltpu.sync_copy(d1_hbm.at[pl.ds(bc, _CHUNK)], i1)
            pltpu.async_copy(ys_hbm.at[i0], b0, sem).wait()
            pltpu.async_copy(ys_hbm.at[i1], b1, sem).wait()
            pltpu.sync_copy(b0, r0_hbm.at[pl.ds(bc, _CHUNK)])
            pltpu.sync_copy(b1, r1_hbm.at[pl.ds(bc, _CHUNK)])

    return _combine_sc


def kernel(x, gate_w, w1, b1, w2, b2):
    b, s, hd = x.shape
    e, i, _ = w1.shape
    xf = x.reshape(s, hd)

    w0s, w1s, d0, d1, tstart, ntile = pl.pallas_call(
        _route_body,
        grid=(1,),
        in_specs=[
            pl.BlockSpec((s, hd), lambda _: (0, 0)),
            pl.BlockSpec((e, hd), lambda _: (0, 0)),
        ],
        out_specs=[
            pl.BlockSpec((s, 1), lambda _: (0, 0)),
            pl.BlockSpec((s, 1), lambda _: (0, 0)),
            pl.BlockSpec((s, 1), lambda _: (0, 0)),
            pl.BlockSpec((s, 1), lambda _: (0, 0)),
            pl.BlockSpec((e, 1), lambda _: (0, 0)),
            pl.BlockSpec((e, 1), lambda _: (0, 0)),
        ],
        out_shape=[
            jax.ShapeDtypeStruct((s, 1), jnp.float32),
            jax.ShapeDtypeStruct((s, 1), jnp.float32),
            jax.ShapeDtypeStruct((s, 1), jnp.int32),
            jax.ShapeDtypeStruct((s, 1), jnp.int32),
            jax.ShapeDtypeStruct((e, 1), jnp.int32),
            jax.ShapeDtypeStruct((e, 1), jnp.int32),
        ],
    )(xf, gate_w)

    d0f = d0.reshape(s)
    d1f = d1.reshape(s)
    xs = _make_dispatch_sc()(xf, d0f, d1f)

    b1r = b1.reshape(e, _NIC, 1, _ICS)
    b2r = b2.reshape(e, 1, hd)

    ys = pl.pallas_call(
        _ffn_body,
        grid_spec=pltpu.PrefetchScalarGridSpec(
            num_scalar_prefetch=2,
            grid=(e, _NIC),
            in_specs=[
                pl.BlockSpec(memory_space=pl.ANY),
                pl.BlockSpec((1, _ICS, hd),
                             lambda ei, ic, ts, nt: (ei, ic, 0)),
                pl.BlockSpec((1, 1, 1, _ICS),
                             lambda ei, ic, ts, nt: (ei, ic, 0, 0)),
                pl.BlockSpec((1, hd, _ICS),
                             lambda ei, ic, ts, nt: (ei, 0, ic)),
                pl.BlockSpec((1, 1, hd),
                             lambda ei, ic, ts, nt: (ei, 0, 0)),
            ],
            out_specs=pl.BlockSpec(memory_space=pl.ANY),
            scratch_shapes=[
                pltpu.VMEM((_T, hd), jnp.float32),
                pltpu.VMEM((_T, hd), jnp.float32),
                pltpu.VMEM((_MAXT, _T, hd), jnp.float32),
                pltpu.SemaphoreType.DMA((2,)),
                pltpu.SemaphoreType.DMA((_MAXT,)),
            ],
        ),
        out_shape=jax.ShapeDtypeStruct((_NROWS, hd), jnp.float32),
        compiler_params=pltpu.CompilerParams(
            dimension_semantics=("arbitrary", "arbitrary")),
    )(tstart.reshape(e), ntile.reshape(e), xs, w1, b1r, w2, b2r)

    r0, r1 = _make_combine_sc()(ys, d0f, d1f)

    ts = 512
    out = pl.pallas_call(
        _comb_body,
        grid=(s // ts,),
        in_specs=[
            pl.BlockSpec((ts, hd), lambda si: (si, 0)),
            pl.BlockSpec((ts, hd), lambda si: (si, 0)),
            pl.BlockSpec((ts, 1), lambda si: (si, 0)),
            pl.BlockSpec((ts, 1), lambda si: (si, 0)),
        ],
        out_specs=pl.BlockSpec((ts, hd), lambda si: (si, 0)),
        out_shape=jax.ShapeDtypeStruct((s, hd), jnp.float32),
    )(r0, r1, w0s, w1s)
    return out.reshape(b, s, hd)
